# Initial kernel scaffold; baseline (speedup 1.0000x reference)
import functools

import jax
import jax.numpy as jnp
from jax import lax
from jax.experimental import pallas as pl
from jax.experimental.pallas import tpu as pltpu
from jax.experimental.pallas import tpu_sc as plsc

N = 10000
E = 320000
D = 128
H = 512
C = 40

NC = 2
NS = 16
NW = NC * NS
L = 16

NP = 10240
NODES_PER_TILE = NP // NS
NVEC = NODES_PER_TILE // L

CHUNK = 128
EDGES_PER_WORKER = ((E // NW + CHUNK - 1) // CHUNK) * CHUNK
EP = EDGES_PER_WORKER * NW
AGG_CHUNKS = EDGES_PER_WORKER // CHUNK
EDGES_PER_TILE_DEG = EP // NS
DEG_CHUNKS = EDGES_PER_TILE_DEG // CHUNK

CPAD = 128


def _rsqrt16(d):
  i = plsc.bitcast(d, jnp.int32)
  y = plsc.bitcast(jnp.int32(0x5F3759DF) - (i >> 1), jnp.float32)
  for _ in range(3):
    y = y * (jnp.float32(1.5) - jnp.float32(0.5) * d * y * y)
  return y


def _sc_body(x_hbm, row_hbm, col_hbm, ew_hbm, part_hbm, deg_hbm,
             agg_sh, degp_sh, deg_sh,
             dis_v, tmp_v, degslice_v, row_v, col_v, ew_v, norm_v, rows_v):
  cid = lax.axis_index("c")
  sid = lax.axis_index("s")
  wid = cid * NS + sid
  zero16 = jnp.zeros((L,), jnp.float32)

  @pl.loop(0, NP // L)
  def _(i):
    dis_v[pl.ds(i * L, L)] = zero16

  @pl.loop(0, CHUNK)
  def _(r):
    for g in range(D // L):
      rows_v[r, pl.ds(g * L, L)] = zero16

  for k in range(NODES_PER_TILE // CHUNK):
    pltpu.sync_copy(rows_v, agg_sh.at[pl.ds(sid * NODES_PER_TILE + k * CHUNK,
                                            CHUNK)])

  @pl.loop(0, DEG_CHUNKS)
  def _(c):
    base = sid * EDGES_PER_TILE_DEG + c * CHUNK
    pltpu.sync_copy(col_hbm.at[pl.ds(base, CHUNK)], col_v)
    pltpu.sync_copy(ew_hbm.at[pl.ds(base, CHUNK)], ew_v)
    for g in range(CHUNK // L):
      idx = col_v[pl.ds(g * L, L)]
      w = ew_v[pl.ds(g * L, L)]
      plsc.addupdate_scatter(dis_v, [idx], w)

  pltpu.sync_copy(dis_v, degp_sh.at[sid])
  plsc.subcore_barrier()

  nbase = sid * NODES_PER_TILE
  for j in range(NS):
    pltpu.sync_copy(degp_sh.at[j, pl.ds(nbase, NODES_PER_TILE)], tmp_v.at[j])

  @pl.loop(0, NVEC)
  def _(g):
    acc = jnp.full((L,), 1.0, jnp.float32)
    for j in range(NS):
      acc = acc + tmp_v[j, pl.ds(g * L, L)]
    degslice_v[pl.ds(g * L, L)] = acc

  pltpu.sync_copy(degslice_v, deg_sh.at[pl.ds(nbase, NODES_PER_TILE)])

  @pl.when(cid == 0)
  def _():
    pltpu.sync_copy(degslice_v, deg_hbm.at[pl.ds(nbase, NODES_PER_TILE)])

  plsc.subcore_barrier()

  pltpu.sync_copy(deg_sh, dis_v)

  @pl.loop(0, NP // L)
  def _(i):
    d = dis_v[pl.ds(i * L, L)]
    dis_v[pl.ds(i * L, L)] = _rsqrt16(d)

  @pl.loop(0, AGG_CHUNKS)
  def _(c):
    base = wid * EDGES_PER_WORKER + c * CHUNK
    pltpu.sync_copy(row_hbm.at[pl.ds(base, CHUNK)], row_v)
    pltpu.sync_copy(col_hbm.at[pl.ds(base, CHUNK)], col_v)
    pltpu.sync_copy(ew_hbm.at[pl.ds(base, CHUNK)], ew_v)
    pltpu.sync_copy(x_hbm.at[row_v], rows_v)

    for g in range(CHUNK // L):
      r16 = row_v[pl.ds(g * L, L)]
      disr = plsc.load_gather(dis_v, [r16])
      norm_v[pl.ds(g * L, L)] = ew_v[pl.ds(g * L, L)] * disr

    @pl.loop(0, CHUNK)
    def _(j):
      s = norm_v[j]
      for g in range(D // L):
        rows_v[j, pl.ds(g * L, L)] = rows_v[j, pl.ds(g * L, L)] * s

    pltpu.sync_copy(rows_v, agg_sh.at[col_v], add=True)

  plsc.subcore_barrier()

  pltpu.sync_copy(agg_sh.at[pl.ds(nbase, NODES_PER_TILE)],
                  part_hbm.at[cid, pl.ds(nbase, NODES_PER_TILE)])


@jax.jit
def _sc_aggregate(x_pad, row_p, col_p, ew_p):
  mesh = plsc.VectorSubcoreMesh(core_axis_name="c", subcore_axis_name="s")
  k = pl.kernel(
      _sc_body,
      out_type=(
          jax.ShapeDtypeStruct((NC, NP, D), jnp.float32),
          jax.ShapeDtypeStruct((NP,), jnp.float32),
      ),
      mesh=mesh,
      scratch_types=[
          pltpu.VMEM_SHARED((NP, D), jnp.float32),
          pltpu.VMEM_SHARED((NS, NP), jnp.float32),
          pltpu.VMEM_SHARED((NP,), jnp.float32),
          pltpu.VMEM((NP,), jnp.float32),
          pltpu.VMEM((NS, NODES_PER_TILE), jnp.float32),
          pltpu.VMEM((NODES_PER_TILE,), jnp.float32),
          pltpu.VMEM((CHUNK,), jnp.int32),
          pltpu.VMEM((CHUNK,), jnp.int32),
          pltpu.VMEM((CHUNK,), jnp.float32),
          pltpu.VMEM((CHUNK,), jnp.float32),
          pltpu.VMEM((CHUNK, D), jnp.float32),
      ],
  )
  return k(x_pad, row_p, col_p, ew_p)


def _tc_body(part_ref, x_ref, deg_ref, wc_ref, bc_ref, w1_ref, b1_ref,
             w2_ref, b2_ref, out_ref):
  deg = deg_ref[...]
  dis = lax.rsqrt(deg)
  agg = (part_ref[0] + part_ref[1]) * dis + x_ref[...] * (1.0 / deg)
  h = jnp.dot(agg, wc_ref[...], preferred_element_type=jnp.float32)
  h = jnp.maximum(h + bc_ref[...], 0.0)
  h = jnp.dot(h, w1_ref[...], preferred_element_type=jnp.float32)
  h = jnp.maximum(h + b1_ref[...], 0.0)
  logits = jnp.dot(h, w2_ref[...], preferred_element_type=jnp.float32)
  logits = logits + b2_ref[...]
  valid = lax.broadcasted_iota(jnp.int32, logits.shape, 1) < C
  masked = jnp.where(valid, logits, -jnp.inf)
  m = jnp.max(masked, axis=1, keepdims=True)
  s = jnp.sum(jnp.where(valid, jnp.exp(logits - m), 0.0), axis=1,
              keepdims=True)
  out_ref[...] = logits - m - jnp.log(s)


@jax.jit
def _tc_head(part, x_pad, deg2, W_conv, b_conv2, W_fc1, b_fc12, W_fc2p,
             b_fc2p2):
  BN = 256
  grid = (NP // BN,)
  return pl.pallas_call(
      _tc_body,
      grid=grid,
      in_specs=[
          pl.BlockSpec((NC, BN, D), lambda i: (0, i, 0)),
          pl.BlockSpec((BN, D), lambda i: (i, 0)),
          pl.BlockSpec((BN, 1), lambda i: (i, 0)),
          pl.BlockSpec((D, H), lambda i: (0, 0)),
          pl.BlockSpec((1, H), lambda i: (0, 0)),
          pl.BlockSpec((H, H), lambda i: (0, 0)),
          pl.BlockSpec((1, H), lambda i: (0, 0)),
          pl.BlockSpec((H, CPAD), lambda i: (0, 0)),
          pl.BlockSpec((1, CPAD), lambda i: (0, 0)),
      ],
      out_specs=pl.BlockSpec((BN, CPAD), lambda i: (i, 0)),
      out_shape=jax.ShapeDtypeStruct((NP, CPAD), jnp.float32),
  )(part, x_pad, deg2, W_conv, b_conv2, W_fc1, b_fc12, W_fc2p, b_fc2p2)


def kernel(x, edge_index, edge_attr, W_conv, b_conv, W_fc1, b_fc1, W_fc2,
           b_fc2):
  row = jnp.pad(edge_index[0], (0, EP - E))
  col = jnp.pad(edge_index[1], (0, EP - E))
  ew = jnp.pad(jnp.squeeze(edge_attr, -1), (0, EP - E))
  x_pad = jnp.pad(x, ((0, NP - N), (0, 0)))

  part, deg = _sc_aggregate(x_pad, row, col, ew)

  W_fc2p = jnp.pad(W_fc2, ((0, 0), (0, CPAD - C)))
  b_fc2p = jnp.pad(b_fc2, (0, CPAD - C))
  out = _tc_head(part, x_pad, deg.reshape(NP, 1), W_conv,
                 b_conv.reshape(1, H), W_fc1, b_fc1.reshape(1, H), W_fc2p,
                 b_fc2p.reshape(1, CPAD))
  return out[:N, :C]


# speedup vs baseline: 15.0705x; 15.0705x over previous
import functools

import jax
import jax.numpy as jnp
from jax import lax
from jax.experimental import pallas as pl
from jax.experimental.pallas import tpu as pltpu
from jax.experimental.pallas import tpu_sc as plsc

N = 10000
E = 320000
D = 128
H = 512
C = 40

NC = 2
NS = 16
NW = NC * NS
L = 16

NP = 10240
NODES_PER_TILE = NP // NS
NVEC = NODES_PER_TILE // L

CHUNK = 128
EDGES_PER_WORKER = ((E // NW + CHUNK - 1) // CHUNK) * CHUNK
EP = EDGES_PER_WORKER * NW
AGG_CHUNKS = EDGES_PER_WORKER // CHUNK
EDGES_PER_TILE_DEG = EP // NS
DEG_CHUNKS = EDGES_PER_TILE_DEG // CHUNK

CPAD = 128


def _rsqrt16(d):
  i = plsc.bitcast(d, jnp.int32)
  y = plsc.bitcast(jnp.int32(0x5F3759DF) - (i >> 1), jnp.float32)
  for _ in range(3):
    y = y * (jnp.float32(1.5) - jnp.float32(0.5) * d * y * y)
  return y


def _sc_body(x_hbm, row_hbm, col_hbm, ew_hbm, part_hbm, deg_hbm,
             agg_sh, degp_sh, deg_sh,
             dis_v, tmp_v, degslice_v, row_v, col_v, ew_v, norm_v, rows_v):
  cid = lax.axis_index("c")
  sid = lax.axis_index("s")
  wid = cid * NS + sid
  zero16 = jnp.zeros((L,), jnp.float32)

  @pl.loop(0, NP // L)
  def _(i):
    dis_v[pl.ds(i * L, L)] = zero16

  @pl.loop(0, CHUNK)
  def _(r):
    for g in range(D // L):
      rows_v[r, pl.ds(g * L, L)] = zero16

  for k in range(NODES_PER_TILE // CHUNK):
    pltpu.sync_copy(rows_v, agg_sh.at[pl.ds(sid * NODES_PER_TILE + k * CHUNK,
                                            CHUNK)])

  @pl.loop(0, DEG_CHUNKS)
  def _(c):
    base = sid * EDGES_PER_TILE_DEG + c * CHUNK
    pltpu.sync_copy(col_hbm.at[pl.ds(base, CHUNK)], col_v)
    pltpu.sync_copy(ew_hbm.at[pl.ds(base, CHUNK)], ew_v)
    for g in range(CHUNK // L):
      idx = col_v[pl.ds(g * L, L)]
      w = ew_v[pl.ds(g * L, L)]
      plsc.addupdate_scatter(dis_v, [idx], w)

  pltpu.sync_copy(dis_v, degp_sh.at[sid])
  plsc.subcore_barrier()

  nbase = sid * NODES_PER_TILE
  for j in range(NS):
    pltpu.sync_copy(degp_sh.at[j, pl.ds(nbase, NODES_PER_TILE)], tmp_v.at[j])

  @pl.loop(0, NVEC)
  def _(g):
    acc = jnp.full((L,), 1.0, jnp.float32)
    for j in range(NS):
      acc = acc + tmp_v[j, pl.ds(g * L, L)]
    degslice_v[pl.ds(g * L, L)] = acc

  pltpu.sync_copy(degslice_v, deg_sh.at[pl.ds(nbase, NODES_PER_TILE)])

  @pl.when(cid == 0)
  def _():
    pltpu.sync_copy(degslice_v, deg_hbm.at[pl.ds(nbase, NODES_PER_TILE)])

  plsc.subcore_barrier()

  pltpu.sync_copy(deg_sh, dis_v)

  @pl.loop(0, NP // L)
  def _(i):
    d = dis_v[pl.ds(i * L, L)]
    dis_v[pl.ds(i * L, L)] = _rsqrt16(d)

  @pl.loop(0, AGG_CHUNKS)
  def _(c):
    base = wid * EDGES_PER_WORKER + c * CHUNK
    pltpu.sync_copy(row_hbm.at[pl.ds(base, CHUNK)], row_v)
    pltpu.sync_copy(col_hbm.at[pl.ds(base, CHUNK)], col_v)
    pltpu.sync_copy(ew_hbm.at[pl.ds(base, CHUNK)], ew_v)
    pltpu.sync_copy(x_hbm.at[row_v], rows_v)

    for g in range(CHUNK // L):
      r16 = row_v[pl.ds(g * L, L)]
      disr = plsc.load_gather(dis_v, [r16])
      norm_v[pl.ds(g * L, L)] = ew_v[pl.ds(g * L, L)] * disr

    @pl.loop(0, CHUNK // L)
    def _(q):
      nv = norm_v[pl.ds(q * L, L)]
      for lane in range(L):
        j = q * L + lane
        s = nv[lane]
        for g in range(D // L):
          rows_v[j, pl.ds(g * L, L)] = rows_v[j, pl.ds(g * L, L)] * s

    pltpu.sync_copy(rows_v, agg_sh.at[col_v], add=True)

  plsc.subcore_barrier()

  pltpu.sync_copy(agg_sh.at[pl.ds(nbase, NODES_PER_TILE)],
                  part_hbm.at[cid, pl.ds(nbase, NODES_PER_TILE)])


@jax.jit
def _sc_aggregate(x_pad, row_p, col_p, ew_p):
  mesh = plsc.VectorSubcoreMesh(core_axis_name="c", subcore_axis_name="s")
  k = pl.kernel(
      _sc_body,
      out_type=(
          jax.ShapeDtypeStruct((NC, NP, D), jnp.float32),
          jax.ShapeDtypeStruct((NP,), jnp.float32),
      ),
      mesh=mesh,
      compiler_params=pltpu.CompilerParams(needs_layout_passes=False),
      scratch_types=[
          pltpu.VMEM_SHARED((NP, D), jnp.float32),
          pltpu.VMEM_SHARED((NS, NP), jnp.float32),
          pltpu.VMEM_SHARED((NP,), jnp.float32),
          pltpu.VMEM((NP,), jnp.float32),
          pltpu.VMEM((NS, NODES_PER_TILE), jnp.float32),
          pltpu.VMEM((NODES_PER_TILE,), jnp.float32),
          pltpu.VMEM((CHUNK,), jnp.int32),
          pltpu.VMEM((CHUNK,), jnp.int32),
          pltpu.VMEM((CHUNK,), jnp.float32),
          pltpu.VMEM((CHUNK,), jnp.float32),
          pltpu.VMEM((CHUNK, D), jnp.float32),
      ],
  )
  return k(x_pad, row_p, col_p, ew_p)


def _tc_body(part_ref, x_ref, deg_ref, wc_ref, bc_ref, w1_ref, b1_ref,
             w2_ref, b2_ref, out_ref):
  deg = deg_ref[...]
  dis = lax.rsqrt(deg)
  agg = (part_ref[0] + part_ref[1]) * dis + x_ref[...] * (1.0 / deg)
  h = jnp.dot(agg, wc_ref[...], preferred_element_type=jnp.float32)
  h = jnp.maximum(h + bc_ref[...], 0.0)
  h = jnp.dot(h, w1_ref[...], preferred_element_type=jnp.float32)
  h = jnp.maximum(h + b1_ref[...], 0.0)
  logits = jnp.dot(h, w2_ref[...], preferred_element_type=jnp.float32)
  logits = logits + b2_ref[...]
  valid = lax.broadcasted_iota(jnp.int32, logits.shape, 1) < C
  masked = jnp.where(valid, logits, -jnp.inf)
  m = jnp.max(masked, axis=1, keepdims=True)
  s = jnp.sum(jnp.where(valid, jnp.exp(logits - m), 0.0), axis=1,
              keepdims=True)
  out_ref[...] = logits - m - jnp.log(s)


@jax.jit
def _tc_head(part, x_pad, deg2, W_conv, b_conv2, W_fc1, b_fc12, W_fc2p,
             b_fc2p2):
  BN = 256
  grid = (NP // BN,)
  return pl.pallas_call(
      _tc_body,
      grid=grid,
      in_specs=[
          pl.BlockSpec((NC, BN, D), lambda i: (0, i, 0)),
          pl.BlockSpec((BN, D), lambda i: (i, 0)),
          pl.BlockSpec((BN, 1), lambda i: (i, 0)),
          pl.BlockSpec((D, H), lambda i: (0, 0)),
          pl.BlockSpec((1, H), lambda i: (0, 0)),
          pl.BlockSpec((H, H), lambda i: (0, 0)),
          pl.BlockSpec((1, H), lambda i: (0, 0)),
          pl.BlockSpec((H, CPAD), lambda i: (0, 0)),
          pl.BlockSpec((1, CPAD), lambda i: (0, 0)),
      ],
      out_specs=pl.BlockSpec((BN, CPAD), lambda i: (i, 0)),
      out_shape=jax.ShapeDtypeStruct((NP, CPAD), jnp.float32),
  )(part, x_pad, deg2, W_conv, b_conv2, W_fc1, b_fc12, W_fc2p, b_fc2p2)


def kernel(x, edge_index, edge_attr, W_conv, b_conv, W_fc1, b_fc1, W_fc2,
           b_fc2):
  row = jnp.pad(edge_index[0], (0, EP - E))
  col = jnp.pad(edge_index[1], (0, EP - E))
  ew = jnp.pad(jnp.squeeze(edge_attr, -1), (0, EP - E))
  x_pad = jnp.pad(x, ((0, NP - N), (0, 0)))

  part, deg = _sc_aggregate(x_pad, row, col, ew)

  W_fc2p = jnp.pad(W_fc2, ((0, 0), (0, CPAD - C)))
  b_fc2p = jnp.pad(b_fc2, (0, CPAD - C))
  out = _tc_head(part, x_pad, deg.reshape(NP, 1), W_conv,
                 b_conv.reshape(1, H), W_fc1, b_fc1.reshape(1, H), W_fc2p,
                 b_fc2p.reshape(1, CPAD))
  return out[:N, :C]


# speedup vs baseline: 15.7853x; 1.0474x over previous
import jax
import jax.numpy as jnp
from jax import lax
from jax.experimental import pallas as pl
from jax.experimental.pallas import tpu as pltpu
from jax.experimental.pallas import tpu_sc as plsc

N = 10000
E = 320000
D = 128
H = 512
C = 40

NC = 2
NS = 16
NW = NC * NS
L = 16

NP = 10240
NODES_PER_TILE = NP // NS
NODES_PER_WORKER = NP // NW
NVEC = NODES_PER_TILE // L

CHUNK = 128
AGG_CHUNKS = 80
EDGES_PER_WORKER = AGG_CHUNKS * CHUNK
EP = EDGES_PER_WORKER * NW
ECH = EP // CHUNK
DEG_CHUNKS = ECH // NS

BLK = 16
NBLOCKS = AGG_CHUNKS // BLK
XROWS = 64

CPAD = 128


def _rsqrt16(d):
  i = plsc.bitcast(d, jnp.int32)
  y = plsc.bitcast(jnp.int32(0x5F3759DF) - (i >> 1), jnp.float32)
  for _ in range(3):
    y = y * (jnp.float32(1.5) - jnp.float32(0.5) * d * y * y)
  return y


def _deg_body(x_hbm, col_hbm, ew_hbm, deg_hbm, xs_hbm,
              degp_sh, deg_sh,
              dega_v, colD_v, ewD_v, tmp_v, degslice_v, diss_v, xbuf_v):
  cid = lax.axis_index("c")
  sid = lax.axis_index("s")
  wid = cid * NS + sid
  zero16 = jnp.zeros((L,), jnp.float32)

  @pl.loop(0, NP // L)
  def _(i):
    dega_v[pl.ds(i * L, L)] = zero16

  pltpu.sync_copy(col_hbm.at[pl.ds(sid * DEG_CHUNKS, DEG_CHUNKS)], colD_v)
  pltpu.sync_copy(ew_hbm.at[pl.ds(sid * DEG_CHUNKS, DEG_CHUNKS)], ewD_v)

  @pl.loop(0, DEG_CHUNKS)
  def _(c):
    for g in range(CHUNK // L):
      idx = colD_v[c, pl.ds(g * L, L)]
      w = ewD_v[c, pl.ds(g * L, L)]
      plsc.addupdate_scatter(dega_v, [idx], w)

  pltpu.sync_copy(dega_v, degp_sh.at[sid])
  plsc.subcore_barrier()

  nbase = sid * NODES_PER_TILE
  for j in range(NS):
    pltpu.sync_copy(degp_sh.at[j, pl.ds(nbase, NODES_PER_TILE)], tmp_v.at[j])

  @pl.loop(0, NVEC)
  def _(g):
    acc = jnp.full((L,), 1.0, jnp.float32)
    for j in range(NS):
      acc = acc + tmp_v[j, pl.ds(g * L, L)]
    degslice_v[pl.ds(g * L, L)] = acc

  pltpu.sync_copy(degslice_v, deg_sh.at[pl.ds(nbase, NODES_PER_TILE)])

  @pl.when(cid == 0)
  def _():
    pltpu.sync_copy(degslice_v, deg_hbm.at[pl.ds(nbase, NODES_PER_TILE)])

  plsc.subcore_barrier()

  rbase = wid * NODES_PER_WORKER
  pltpu.sync_copy(deg_sh.at[pl.ds(rbase, NODES_PER_WORKER)], diss_v)

  @pl.loop(0, NODES_PER_WORKER // L)
  def _(i):
    diss_v[pl.ds(i * L, L)] = _rsqrt16(diss_v[pl.ds(i * L, L)])

  @pl.loop(0, NODES_PER_WORKER // XROWS)
  def _(b):
    pltpu.sync_copy(x_hbm.at[pl.ds(rbase + b * XROWS, XROWS)], xbuf_v)

    @pl.loop(0, XROWS // L)
    def _(q):
      dv = diss_v[pl.ds(b * XROWS + q * L, L)]
      for lane in range(L):
        r = q * L + lane
        s = dv[lane]
        for g in range(D // L):
          xbuf_v[r, pl.ds(g * L, L)] = xbuf_v[r, pl.ds(g * L, L)] * s

    pltpu.sync_copy(xbuf_v, xs_hbm.at[pl.ds(rbase + b * XROWS, XROWS)])


@jax.jit
def _sc_degree(x_pad, col2d, ew2d):
  mesh = plsc.VectorSubcoreMesh(core_axis_name="c", subcore_axis_name="s")
  k = pl.kernel(
      _deg_body,
      out_type=(
          jax.ShapeDtypeStruct((NP,), jnp.float32),
          jax.ShapeDtypeStruct((NP, D), jnp.float32),
      ),
      mesh=mesh,
      compiler_params=pltpu.CompilerParams(needs_layout_passes=False),
      scratch_types=[
          pltpu.VMEM_SHARED((NS, NP), jnp.float32),
          pltpu.VMEM_SHARED((NP,), jnp.float32),
          pltpu.VMEM((NP,), jnp.float32),
          pltpu.VMEM((DEG_CHUNKS, CHUNK), jnp.int32),
          pltpu.VMEM((DEG_CHUNKS, CHUNK), jnp.float32),
          pltpu.VMEM((NS, NODES_PER_TILE), jnp.float32),
          pltpu.VMEM((NODES_PER_TILE,), jnp.float32),
          pltpu.VMEM((NODES_PER_WORKER,), jnp.float32),
          pltpu.VMEM((XROWS, D), jnp.float32),
      ],
  )
  return k(x_pad, col2d, ew2d)


def _agg_body(xs_hbm, row_hbm, col_hbm, ew_hbm, part_hbm,
              agg_sh, rowB_v, colB_v, ewB_v, rows2_v,
              gsem0, gsem1, ssem0, ssem1):
  cid = lax.axis_index("c")
  sid = lax.axis_index("s")
  wid = cid * NS + sid
  zero16 = jnp.zeros((L,), jnp.float32)

  @pl.loop(0, CHUNK)
  def _(r):
    for g in range(D // L):
      rows2_v[0, r, pl.ds(g * L, L)] = zero16

  for k in range(NODES_PER_TILE // CHUNK):
    pltpu.sync_copy(rows2_v.at[0],
                    agg_sh.at[pl.ds(sid * NODES_PER_TILE + k * CHUNK, CHUNK)])

  plsc.subcore_barrier()

  def scale(b, c):
    @pl.loop(0, CHUNK // L)
    def _(q):
      nv = ewB_v[c, pl.ds(q * L, L)]
      for lane in range(L):
        j = q * L + lane
        s = nv[lane]
        for g in range(D // L):
          rows2_v[b, j, pl.ds(g * L, L)] = rows2_v[b, j, pl.ds(g * L, L)] * s

  @pl.loop(0, NBLOCKS)
  def _(blk):
    cb = wid * AGG_CHUNKS + blk * BLK
    pltpu.sync_copy(row_hbm.at[pl.ds(cb, BLK)], rowB_v)
    pltpu.sync_copy(col_hbm.at[pl.ds(cb, BLK)], colB_v)
    pltpu.sync_copy(ew_hbm.at[pl.ds(cb, BLK)], ewB_v)

    pltpu.async_copy(xs_hbm.at[rowB_v.at[0]], rows2_v.at[0], gsem0)
    pltpu.async_copy(xs_hbm.at[rowB_v.at[1]], rows2_v.at[1], gsem1)

    @pl.loop(0, BLK // 2)
    def _(p):
      c0 = 2 * p
      c1 = 2 * p + 1
      pltpu.make_async_copy(xs_hbm.at[rowB_v.at[c0]], rows2_v.at[0],
                            gsem0).wait()
      scale(0, c0)
      s0 = pltpu.async_copy(rows2_v.at[0], agg_sh.at[colB_v.at[c0]], ssem0,
                            add=True)
      pltpu.make_async_copy(xs_hbm.at[rowB_v.at[c1]], rows2_v.at[1],
                            gsem1).wait()
      scale(1, c1)
      s1 = pltpu.async_copy(rows2_v.at[1], agg_sh.at[colB_v.at[c1]], ssem1,
                            add=True)
      c2 = jnp.minimum(c0 + 2, BLK - 1)
      c3 = jnp.minimum(c1 + 2, BLK - 1)
      s0.wait()

      @pl.when(p < BLK // 2 - 1)
      def _():
        pltpu.async_copy(xs_hbm.at[rowB_v.at[c2]], rows2_v.at[0], gsem0)

      s1.wait()

      @pl.when(p < BLK // 2 - 1)
      def _():
        pltpu.async_copy(xs_hbm.at[rowB_v.at[c3]], rows2_v.at[1], gsem1)

  plsc.subcore_barrier()

  nbase = sid * NODES_PER_TILE
  pltpu.sync_copy(agg_sh.at[pl.ds(nbase, NODES_PER_TILE)],
                  part_hbm.at[cid, pl.ds(nbase, NODES_PER_TILE)])


@jax.jit
def _sc_scatter(xs, row2d, col2d, ew2d):
  mesh = plsc.VectorSubcoreMesh(core_axis_name="c", subcore_axis_name="s")
  k = pl.kernel(
      _agg_body,
      out_type=jax.ShapeDtypeStruct((NC, NP, D), jnp.float32),
      mesh=mesh,
      compiler_params=pltpu.CompilerParams(needs_layout_passes=False),
      scratch_types=[
          pltpu.VMEM_SHARED((NP, D), jnp.float32),
          pltpu.VMEM((BLK, CHUNK), jnp.int32),
          pltpu.VMEM((BLK, CHUNK), jnp.int32),
          pltpu.VMEM((BLK, CHUNK), jnp.float32),
          pltpu.VMEM((2, CHUNK, D), jnp.float32),
          pltpu.SemaphoreType.DMA,
          pltpu.SemaphoreType.DMA,
          pltpu.SemaphoreType.DMA,
          pltpu.SemaphoreType.DMA,
      ],
  )
  return k(xs, row2d, col2d, ew2d)


def _tc_body(part_ref, x_ref, deg_ref, wc_ref, bc_ref, w1_ref, b1_ref,
             w2_ref, b2_ref, out_ref):
  deg = deg_ref[...]
  dis = lax.rsqrt(deg)
  agg = (part_ref[0] + part_ref[1]) * dis + x_ref[...] * (1.0 / deg)
  h = jnp.dot(agg, wc_ref[...], preferred_element_type=jnp.float32)
  h = jnp.maximum(h + bc_ref[...], 0.0)
  h = jnp.dot(h, w1_ref[...], preferred_element_type=jnp.float32)
  h = jnp.maximum(h + b1_ref[...], 0.0)
  logits = jnp.dot(h, w2_ref[...], preferred_element_type=jnp.float32)
  logits = logits + b2_ref[...]
  valid = lax.broadcasted_iota(jnp.int32, logits.shape, 1) < C
  masked = jnp.where(valid, logits, -jnp.inf)
  m = jnp.max(masked, axis=1, keepdims=True)
  s = jnp.sum(jnp.where(valid, jnp.exp(logits - m), 0.0), axis=1,
              keepdims=True)
  out_ref[...] = logits - m - jnp.log(s)


@jax.jit
def _tc_head(part, x_pad, deg2, W_conv, b_conv2, W_fc1, b_fc12, W_fc2p,
             b_fc2p2):
  BN = 256
  grid = (NP // BN,)
  return pl.pallas_call(
      _tc_body,
      grid=grid,
      in_specs=[
          pl.BlockSpec((NC, BN, D), lambda i: (0, i, 0)),
          pl.BlockSpec((BN, D), lambda i: (i, 0)),
          pl.BlockSpec((BN, 1), lambda i: (i, 0)),
          pl.BlockSpec((D, H), lambda i: (0, 0)),
          pl.BlockSpec((1, H), lambda i: (0, 0)),
          pl.BlockSpec((H, H), lambda i: (0, 0)),
          pl.BlockSpec((1, H), lambda i: (0, 0)),
          pl.BlockSpec((H, CPAD), lambda i: (0, 0)),
          pl.BlockSpec((1, CPAD), lambda i: (0, 0)),
      ],
      out_specs=pl.BlockSpec((BN, CPAD), lambda i: (i, 0)),
      out_shape=jax.ShapeDtypeStruct((NP, CPAD), jnp.float32),
  )(part, x_pad, deg2, W_conv, b_conv2, W_fc1, b_fc12, W_fc2p, b_fc2p2)


def kernel(x, edge_index, edge_attr, W_conv, b_conv, W_fc1, b_fc1, W_fc2,
           b_fc2):
  row = jnp.pad(edge_index[0], (0, EP - E)).reshape(ECH, CHUNK)
  col = jnp.pad(edge_index[1], (0, EP - E)).reshape(ECH, CHUNK)
  ew = jnp.pad(jnp.squeeze(edge_attr, -1), (0, EP - E)).reshape(ECH, CHUNK)
  x_pad = jnp.pad(x, ((0, NP - N), (0, 0)))

  deg, xs = _sc_degree(x_pad, col, ew)
  part = _sc_scatter(xs, row, col, ew)

  W_fc2p = jnp.pad(W_fc2, ((0, 0), (0, CPAD - C)))
  b_fc2p = jnp.pad(b_fc2, (0, CPAD - C))
  out = _tc_head(part, x_pad, deg.reshape(NP, 1), W_conv,
                 b_conv.reshape(1, H), W_fc1, b_fc1.reshape(1, H), W_fc2p,
                 b_fc2p.reshape(1, CPAD))
  return out[:N, :C]


# speedup vs baseline: 38.4287x; 2.4345x over previous
import jax
import jax.numpy as jnp
from jax import lax
from jax.experimental import pallas as pl
from jax.experimental.pallas import tpu as pltpu
from jax.experimental.pallas import tpu_sc as plsc

N = 10000
E = 320000
D = 128
H = 512
C = 40

NC = 2
NS = 16
NW = NC * NS
L = 16

NP = 10240
NODES_PER_TILE = NP // NS
NODES_PER_WORKER = NP // NW
NVEC = NODES_PER_TILE // L

CHUNK = 128
AGG_CHUNKS = 80
EDGES_PER_WORKER = AGG_CHUNKS * CHUNK
EP = EDGES_PER_WORKER * NW
ECH = EP // CHUNK
DEG_CHUNKS = ECH // NS

BLK = 16
NBLOCKS = AGG_CHUNKS // BLK
XROWS = 64

CPAD = 128


def _rsqrt16(d):
  i = plsc.bitcast(d, jnp.int32)
  y = plsc.bitcast(jnp.int32(0x5F3759DF) - (i >> 1), jnp.float32)
  for _ in range(3):
    y = y * (jnp.float32(1.5) - jnp.float32(0.5) * d * y * y)
  return y


def _deg_body(x_hbm, col_hbm, ew_hbm, deg_hbm, xs_hbm,
              degp_sh, deg_sh,
              dega_v, colD_v, ewD_v, tmp_v, degslice_v, diss_v, xbuf_v):
  cid = lax.axis_index("c")
  sid = lax.axis_index("s")
  wid = cid * NS + sid
  zero16 = jnp.zeros((L,), jnp.float32)

  @pl.loop(0, NP // L)
  def _(i):
    dega_v[pl.ds(i * L, L)] = zero16

  pltpu.sync_copy(col_hbm.at[pl.ds(sid * DEG_CHUNKS, DEG_CHUNKS)], colD_v)
  pltpu.sync_copy(ew_hbm.at[pl.ds(sid * DEG_CHUNKS, DEG_CHUNKS)], ewD_v)

  @pl.loop(0, DEG_CHUNKS)
  def _(c):
    for g in range(CHUNK // L):
      idx = colD_v[c, pl.ds(g * L, L)]
      w = ewD_v[c, pl.ds(g * L, L)]
      plsc.addupdate_scatter(dega_v, [idx], w)

  pltpu.sync_copy(dega_v, degp_sh.at[sid])
  plsc.subcore_barrier()

  nbase = sid * NODES_PER_TILE
  for j in range(NS):
    pltpu.sync_copy(degp_sh.at[j, pl.ds(nbase, NODES_PER_TILE)], tmp_v.at[j])

  @pl.loop(0, NVEC)
  def _(g):
    acc = jnp.full((L,), 1.0, jnp.float32)
    for j in range(NS):
      acc = acc + tmp_v[j, pl.ds(g * L, L)]
    degslice_v[pl.ds(g * L, L)] = acc

  pltpu.sync_copy(degslice_v, deg_sh.at[pl.ds(nbase, NODES_PER_TILE)])

  @pl.when(cid == 0)
  def _():
    pltpu.sync_copy(degslice_v, deg_hbm.at[pl.ds(nbase, NODES_PER_TILE)])

  plsc.subcore_barrier()

  rbase = wid * NODES_PER_WORKER
  pltpu.sync_copy(deg_sh.at[pl.ds(rbase, NODES_PER_WORKER)], diss_v)

  @pl.loop(0, NODES_PER_WORKER // L)
  def _(i):
    diss_v[pl.ds(i * L, L)] = _rsqrt16(diss_v[pl.ds(i * L, L)])

  @pl.loop(0, NODES_PER_WORKER // XROWS)
  def _(b):
    pltpu.sync_copy(x_hbm.at[pl.ds(rbase + b * XROWS, XROWS)], xbuf_v)

    @pl.loop(0, XROWS // L)
    def _(q):
      dv = diss_v[pl.ds(b * XROWS + q * L, L)]
      for lane in range(L):
        r = q * L + lane
        s = dv[lane]
        for g in range(D // L):
          xbuf_v[r, pl.ds(g * L, L)] = xbuf_v[r, pl.ds(g * L, L)] * s

    pltpu.sync_copy(xbuf_v, xs_hbm.at[pl.ds(rbase + b * XROWS, XROWS)])


@jax.jit
def _sc_degree(x_pad, col2d, ew2d):
  mesh = plsc.VectorSubcoreMesh(core_axis_name="c", subcore_axis_name="s")
  k = pl.kernel(
      _deg_body,
      out_type=(
          jax.ShapeDtypeStruct((NP,), jnp.float32),
          jax.ShapeDtypeStruct((NP, D), jnp.float32),
      ),
      mesh=mesh,
      compiler_params=pltpu.CompilerParams(needs_layout_passes=False),
      scratch_types=[
          pltpu.VMEM_SHARED((NS, NP), jnp.float32),
          pltpu.VMEM_SHARED((NP,), jnp.float32),
          pltpu.VMEM((NP,), jnp.float32),
          pltpu.VMEM((DEG_CHUNKS, CHUNK), jnp.int32),
          pltpu.VMEM((DEG_CHUNKS, CHUNK), jnp.float32),
          pltpu.VMEM((NS, NODES_PER_TILE), jnp.float32),
          pltpu.VMEM((NODES_PER_TILE,), jnp.float32),
          pltpu.VMEM((NODES_PER_WORKER,), jnp.float32),
          pltpu.VMEM((XROWS, D), jnp.float32),
      ],
  )
  return k(x_pad, col2d, ew2d)


def _agg_body(xs_hbm, row_hbm, col_hbm, ew_hbm, part_hbm,
              agg_sh, rowB_v, colB_v, ewB_v, rows2_v,
              gsem0, gsem1, ssem0, ssem1):
  cid = lax.axis_index("c")
  sid = lax.axis_index("s")
  wid = cid * NS + sid
  zero16 = jnp.zeros((L,), jnp.float32)

  @pl.loop(0, CHUNK)
  def _(r):
    for g in range(D // L):
      rows2_v[0, r, pl.ds(g * L, L)] = zero16

  for k in range(NODES_PER_TILE // CHUNK):
    pltpu.sync_copy(rows2_v.at[0],
                    agg_sh.at[pl.ds(sid * NODES_PER_TILE + k * CHUNK, CHUNK)])

  plsc.subcore_barrier()

  def scale(b, c):
    @pl.loop(0, CHUNK // L)
    def _(q):
      nv = ewB_v[c, pl.ds(q * L, L)]
      for lane in range(L):
        j = q * L + lane
        s = nv[lane]
        for g in range(D // L):
          rows2_v[b, j, pl.ds(g * L, L)] = rows2_v[b, j, pl.ds(g * L, L)] * s

  @pl.loop(0, NBLOCKS)
  def _(blk):
    cb = wid * AGG_CHUNKS + blk * BLK
    pltpu.sync_copy(row_hbm.at[pl.ds(cb, BLK)], rowB_v)
    pltpu.sync_copy(col_hbm.at[pl.ds(cb, BLK)], colB_v)
    pltpu.sync_copy(ew_hbm.at[pl.ds(cb, BLK)], ewB_v)

    pltpu.async_copy(xs_hbm.at[rowB_v.at[0]], rows2_v.at[0], gsem0)
    pltpu.async_copy(xs_hbm.at[rowB_v.at[1]], rows2_v.at[1], gsem1)

    @pl.loop(0, BLK // 2)
    def _(p):
      c0 = 2 * p
      c1 = 2 * p + 1
      pltpu.make_async_copy(xs_hbm.at[rowB_v.at[c0]], rows2_v.at[0],
                            gsem0).wait()
      scale(0, c0)
      s0 = pltpu.async_copy(rows2_v.at[0], agg_sh.at[colB_v.at[c0]], ssem0,
                            add=True)
      pltpu.make_async_copy(xs_hbm.at[rowB_v.at[c1]], rows2_v.at[1],
                            gsem1).wait()
      scale(1, c1)
      s1 = pltpu.async_copy(rows2_v.at[1], agg_sh.at[colB_v.at[c1]], ssem1,
                            add=True)
      c2 = jnp.minimum(c0 + 2, BLK - 1)
      c3 = jnp.minimum(c1 + 2, BLK - 1)
      s0.wait()

      @pl.when(p < BLK // 2 - 1)
      def _():
        pltpu.async_copy(xs_hbm.at[rowB_v.at[c2]], rows2_v.at[0], gsem0)

      s1.wait()

      @pl.when(p < BLK // 2 - 1)
      def _():
        pltpu.async_copy(xs_hbm.at[rowB_v.at[c3]], rows2_v.at[1], gsem1)

  plsc.subcore_barrier()

  nbase = sid * NODES_PER_TILE
  pltpu.sync_copy(agg_sh.at[pl.ds(nbase, NODES_PER_TILE)],
                  part_hbm.at[cid, pl.ds(nbase, NODES_PER_TILE)])


@jax.jit
def _sc_scatter(xs, row2d, col2d, ew2d):
  mesh = plsc.VectorSubcoreMesh(core_axis_name="c", subcore_axis_name="s")
  k = pl.kernel(
      _agg_body,
      out_type=jax.ShapeDtypeStruct((NC, NP, D), jnp.float32),
      mesh=mesh,
      compiler_params=pltpu.CompilerParams(needs_layout_passes=False),
      scratch_types=[
          pltpu.VMEM_SHARED((NP, D), jnp.float32),
          pltpu.VMEM((BLK, CHUNK), jnp.int32),
          pltpu.VMEM((BLK, CHUNK), jnp.int32),
          pltpu.VMEM((BLK, CHUNK), jnp.float32),
          pltpu.VMEM((2, CHUNK, D), jnp.float32),
          pltpu.SemaphoreType.DMA,
          pltpu.SemaphoreType.DMA,
          pltpu.SemaphoreType.DMA,
          pltpu.SemaphoreType.DMA,
      ],
  )
  return k(xs, row2d, col2d, ew2d)


def _tc_body(part_ref, x_ref, deg_ref, wc_ref, bc_ref, w1_ref, b1_ref,
             w2_ref, b2_ref, out_ref):
  deg = deg_ref[...]
  dis = lax.rsqrt(deg)
  agg = (part_ref[0] + part_ref[1]) * dis + x_ref[...] * (1.0 / deg)
  h = jnp.dot(agg, wc_ref[...], preferred_element_type=jnp.float32)
  h = jnp.maximum(h + bc_ref[...], 0.0)
  h = jnp.dot(h, w1_ref[...], preferred_element_type=jnp.float32)
  h = jnp.maximum(h + b1_ref[...], 0.0)
  logits = jnp.dot(h, w2_ref[...], preferred_element_type=jnp.float32)
  logits = logits + b2_ref[...]
  valid = lax.broadcasted_iota(jnp.int32, logits.shape, 1) < C
  masked = jnp.where(valid, logits, -jnp.inf)
  m = jnp.max(masked, axis=1, keepdims=True)
  s = jnp.sum(jnp.where(valid, jnp.exp(logits - m), 0.0), axis=1,
              keepdims=True)
  out_ref[...] = logits - m - jnp.log(s)


@jax.jit
def _tc_head(part, x_pad, deg2, W_conv, b_conv2, W_fc1, b_fc12, W_fc2p,
             b_fc2p2):
  BN = 256
  grid = (NP // BN,)
  return pl.pallas_call(
      _tc_body,
      grid=grid,
      in_specs=[
          pl.BlockSpec((NC, BN, D), lambda i: (0, i, 0)),
          pl.BlockSpec((BN, D), lambda i: (i, 0)),
          pl.BlockSpec((BN, 1), lambda i: (i, 0)),
          pl.BlockSpec((D, H), lambda i: (0, 0)),
          pl.BlockSpec((1, H), lambda i: (0, 0)),
          pl.BlockSpec((H, H), lambda i: (0, 0)),
          pl.BlockSpec((1, H), lambda i: (0, 0)),
          pl.BlockSpec((H, CPAD), lambda i: (0, 0)),
          pl.BlockSpec((1, CPAD), lambda i: (0, 0)),
      ],
      out_specs=pl.BlockSpec((BN, CPAD), lambda i: (i, 0)),
      out_shape=jax.ShapeDtypeStruct((NP, CPAD), jnp.float32),
  )(part, x_pad, deg2, W_conv, b_conv2, W_fc1, b_fc12, W_fc2p, b_fc2p2)


def kernel(x, edge_index, edge_attr, W_conv, b_conv, W_fc1, b_fc1, W_fc2,
           b_fc2):
  pad_spread = (jnp.arange(EP - E, dtype=jnp.int32) % (NP - N)) + N
  row = jnp.concatenate([edge_index[0], pad_spread]).reshape(ECH, CHUNK)
  col = jnp.concatenate([edge_index[1], pad_spread]).reshape(ECH, CHUNK)
  ew = jnp.pad(jnp.squeeze(edge_attr, -1), (0, EP - E)).reshape(ECH, CHUNK)
  x_pad = jnp.pad(x, ((0, NP - N), (0, 0)))

  deg, xs = _sc_degree(x_pad, col, ew)
  part = _sc_scatter(xs, row, col, ew)

  W_fc2p = jnp.pad(W_fc2, ((0, 0), (0, CPAD - C)))
  b_fc2p = jnp.pad(b_fc2, (0, CPAD - C))
  out = _tc_head(part, x_pad, deg.reshape(NP, 1), W_conv,
                 b_conv.reshape(1, H), W_fc1, b_fc1.reshape(1, H), W_fc2p,
                 b_fc2p.reshape(1, CPAD))
  return out[:N, :C]


# speedup vs baseline: 38.6233x; 1.0051x over previous
import jax
import jax.numpy as jnp
from jax import lax
from jax.experimental import pallas as pl
from jax.experimental.pallas import tpu as pltpu
from jax.experimental.pallas import tpu_sc as plsc

N = 10000
E = 320000
D = 128
H = 512
C = 40

NC = 2
NS = 16
NW = NC * NS
L = 16

NP = 10240
NODES_PER_TILE = NP // NS
NODES_PER_WORKER = NP // NW
NVEC = NODES_PER_TILE // L

CHUNK = 128
AGG_CHUNKS = 80
EDGES_PER_WORKER = AGG_CHUNKS * CHUNK
EP = EDGES_PER_WORKER * NW
ECH = EP // CHUNK
DEG_CHUNKS = ECH // NS

BLK = 16
NBLOCKS = AGG_CHUNKS // BLK
XROWS = 64
STG = CHUNK // 2

CPAD = 128


def _rsqrt16(d):
  i = plsc.bitcast(d, jnp.int32)
  y = plsc.bitcast(jnp.int32(0x5F3759DF) - (i >> 1), jnp.float32)
  for _ in range(3):
    y = y * (jnp.float32(1.5) - jnp.float32(0.5) * d * y * y)
  return y


def _deg_body(x_hbm, col_hbm, ew_hbm, deg_hbm, xs_hbm,
              degp_sh, deg_sh,
              dega_v, colD_v, ewD_v, tmp_v, degslice_v, diss_v, xbuf_v):
  cid = lax.axis_index("c")
  sid = lax.axis_index("s")
  wid = cid * NS + sid
  zero16 = jnp.zeros((L,), jnp.float32)

  @pl.loop(0, NP // L)
  def _(i):
    dega_v[pl.ds(i * L, L)] = zero16

  pltpu.sync_copy(col_hbm.at[pl.ds(sid * DEG_CHUNKS, DEG_CHUNKS)], colD_v)
  pltpu.sync_copy(ew_hbm.at[pl.ds(sid * DEG_CHUNKS, DEG_CHUNKS)], ewD_v)

  @pl.loop(0, DEG_CHUNKS)
  def _(c):
    for g in range(CHUNK // L):
      idx = colD_v[c, pl.ds(g * L, L)]
      w = ewD_v[c, pl.ds(g * L, L)]
      plsc.addupdate_scatter(dega_v, [idx], w)

  pltpu.sync_copy(dega_v, degp_sh.at[sid])
  plsc.subcore_barrier()

  nbase = sid * NODES_PER_TILE
  for j in range(NS):
    pltpu.sync_copy(degp_sh.at[j, pl.ds(nbase, NODES_PER_TILE)], tmp_v.at[j])

  @pl.loop(0, NVEC)
  def _(g):
    acc = jnp.full((L,), 1.0, jnp.float32)
    for j in range(NS):
      acc = acc + tmp_v[j, pl.ds(g * L, L)]
    degslice_v[pl.ds(g * L, L)] = acc

  pltpu.sync_copy(degslice_v, deg_sh.at[pl.ds(nbase, NODES_PER_TILE)])

  @pl.when(cid == 0)
  def _():
    pltpu.sync_copy(degslice_v, deg_hbm.at[pl.ds(nbase, NODES_PER_TILE)])

  plsc.subcore_barrier()

  rbase = wid * NODES_PER_WORKER
  pltpu.sync_copy(deg_sh.at[pl.ds(rbase, NODES_PER_WORKER)], diss_v)

  @pl.loop(0, NODES_PER_WORKER // L)
  def _(i):
    diss_v[pl.ds(i * L, L)] = _rsqrt16(diss_v[pl.ds(i * L, L)])

  @pl.loop(0, NODES_PER_WORKER // XROWS)
  def _(b):
    pltpu.sync_copy(x_hbm.at[pl.ds(rbase + b * XROWS, XROWS)], xbuf_v)

    @pl.loop(0, XROWS // L)
    def _(q):
      dv = diss_v[pl.ds(b * XROWS + q * L, L)]
      for lane in range(L):
        r = q * L + lane
        s = dv[lane]
        for g in range(D // L):
          xbuf_v[r, pl.ds(g * L, L)] = xbuf_v[r, pl.ds(g * L, L)] * s

    pltpu.sync_copy(xbuf_v, xs_hbm.at[pl.ds(rbase + b * XROWS, XROWS)])


@jax.jit
def _sc_degree(x_pad, col2d, ew2d):
  mesh = plsc.VectorSubcoreMesh(core_axis_name="c", subcore_axis_name="s")
  k = pl.kernel(
      _deg_body,
      out_type=(
          jax.ShapeDtypeStruct((NP,), jnp.float32),
          jax.ShapeDtypeStruct((NP, D), jnp.float32),
      ),
      mesh=mesh,
      compiler_params=pltpu.CompilerParams(needs_layout_passes=False),
      scratch_types=[
          pltpu.VMEM_SHARED((NS, NP), jnp.float32),
          pltpu.VMEM_SHARED((NP,), jnp.float32),
          pltpu.VMEM((NP,), jnp.float32),
          pltpu.VMEM((DEG_CHUNKS, CHUNK), jnp.int32),
          pltpu.VMEM((DEG_CHUNKS, CHUNK), jnp.float32),
          pltpu.VMEM((NS, NODES_PER_TILE), jnp.float32),
          pltpu.VMEM((NODES_PER_TILE,), jnp.float32),
          pltpu.VMEM((NODES_PER_WORKER,), jnp.float32),
          pltpu.VMEM((XROWS, D), jnp.float32),
      ],
  )
  return k(x_pad, col2d, ew2d)


def _agg_body(xs_hbm, row_hbm, col_hbm, ew_hbm, part_hbm,
              agg_sh, rowB_v, colB_v, ewB_v, rows2_v,
              gsem0, gsem1, ssem0, ssem1):
  cid = lax.axis_index("c")
  sid = lax.axis_index("s")
  wid = cid * NS + sid
  zero16 = jnp.zeros((L,), jnp.float32)

  @pl.loop(0, CHUNK)
  def _(r):
    for g in range(D // L):
      rows2_v[0, r, pl.ds(g * L, L)] = zero16

  for k in range(NODES_PER_TILE // CHUNK):
    pltpu.sync_copy(rows2_v.at[0],
                    agg_sh.at[pl.ds(sid * NODES_PER_TILE + k * CHUNK, CHUNK)])

  plsc.subcore_barrier()

  def scale(b, c):
    @pl.loop(0, CHUNK // L)
    def _(q):
      nv = ewB_v[c, pl.ds(q * L, L)]
      for lane in range(L):
        j = q * L + lane
        s = nv[lane]
        for g in range(D // L):
          rows2_v[b, j, pl.ds(g * L, L)] = rows2_v[b, j, pl.ds(g * L, L)] * s

  @pl.loop(0, NBLOCKS)
  def _(blk):
    cb = wid * AGG_CHUNKS + blk * BLK
    pltpu.sync_copy(row_hbm.at[pl.ds(cb, BLK)], rowB_v)
    pltpu.sync_copy(col_hbm.at[pl.ds(cb, BLK)], colB_v)
    pltpu.sync_copy(ew_hbm.at[pl.ds(cb, BLK)], ewB_v)

    pltpu.async_copy(xs_hbm.at[rowB_v.at[0]], rows2_v.at[0], gsem0)
    pltpu.async_copy(xs_hbm.at[rowB_v.at[1]], rows2_v.at[1], gsem1)

    @pl.loop(0, BLK // 2)
    def _(p):
      c0 = 2 * p
      c1 = 2 * p + 1
      pltpu.make_async_copy(xs_hbm.at[rowB_v.at[c0]], rows2_v.at[0],
                            gsem0).wait()
      scale(0, c0)
      s0 = pltpu.async_copy(rows2_v.at[0], agg_sh.at[colB_v.at[c0]], ssem0,
                            add=True)
      pltpu.make_async_copy(xs_hbm.at[rowB_v.at[c1]], rows2_v.at[1],
                            gsem1).wait()
      scale(1, c1)
      s1 = pltpu.async_copy(rows2_v.at[1], agg_sh.at[colB_v.at[c1]], ssem1,
                            add=True)
      c2 = jnp.minimum(c0 + 2, BLK - 1)
      c3 = jnp.minimum(c1 + 2, BLK - 1)
      s0.wait()

      @pl.when(p < BLK // 2 - 1)
      def _():
        pltpu.async_copy(xs_hbm.at[rowB_v.at[c2]], rows2_v.at[0], gsem0)

      s1.wait()

      @pl.when(p < BLK // 2 - 1)
      def _():
        pltpu.async_copy(xs_hbm.at[rowB_v.at[c3]], rows2_v.at[1], gsem1)

  plsc.subcore_barrier()

  nbase = sid * NODES_PER_TILE
  pltpu.sync_copy(agg_sh.at[pl.ds(nbase, NODES_PER_TILE)],
                  part_hbm.at[cid, pl.ds(nbase, NODES_PER_TILE)])


@jax.jit
def _sc_scatter(xs, row2d, col2d, ew2d):
  mesh = plsc.VectorSubcoreMesh(core_axis_name="c", subcore_axis_name="s")
  k = pl.kernel(
      _agg_body,
      out_type=jax.ShapeDtypeStruct((NC, NP, D), jnp.float32),
      mesh=mesh,
      compiler_params=pltpu.CompilerParams(needs_layout_passes=False),
      scratch_types=[
          pltpu.VMEM_SHARED((NP, D), jnp.float32),
          pltpu.VMEM((BLK, CHUNK), jnp.int32),
          pltpu.VMEM((BLK, CHUNK), jnp.int32),
          pltpu.VMEM((BLK, CHUNK), jnp.float32),
          pltpu.VMEM((2, CHUNK, D), jnp.float32),
          pltpu.SemaphoreType.DMA,
          pltpu.SemaphoreType.DMA,
          pltpu.SemaphoreType.DMA,
          pltpu.SemaphoreType.DMA,
      ],
  )
  return k(xs, row2d, col2d, ew2d)


def _tc_body(part_ref, x_ref, deg_ref, wc_ref, bc_ref, w1_ref, b1_ref,
             w2_ref, b2_ref, out_ref):
  deg = deg_ref[...]
  dis = lax.rsqrt(deg)
  agg = (part_ref[0] + part_ref[1]) * dis + x_ref[...] * (1.0 / deg)
  h = jnp.dot(agg.astype(jnp.bfloat16), wc_ref[...],
              preferred_element_type=jnp.float32)
  h = jnp.maximum(h + bc_ref[...], 0.0)
  h = jnp.dot(h.astype(jnp.bfloat16), w1_ref[...],
              preferred_element_type=jnp.float32)
  h = jnp.maximum(h + b1_ref[...], 0.0)
  logits = jnp.dot(h.astype(jnp.bfloat16), w2_ref[...],
                   preferred_element_type=jnp.float32)
  logits = logits + b2_ref[...]
  valid = lax.broadcasted_iota(jnp.int32, logits.shape, 1) < C
  masked = jnp.where(valid, logits, -jnp.inf)
  m = jnp.max(masked, axis=1, keepdims=True)
  s = jnp.sum(jnp.where(valid, jnp.exp(logits - m), 0.0), axis=1,
              keepdims=True)
  out_ref[...] = logits - m - jnp.log(s)


@jax.jit
def _tc_head(part, x_pad, deg2, W_conv, b_conv2, W_fc1, b_fc12, W_fc2p,
             b_fc2p2):
  BN = 256
  grid = (NP // BN,)
  return pl.pallas_call(
      _tc_body,
      grid=grid,
      in_specs=[
          pl.BlockSpec((NC, BN, D), lambda i: (0, i, 0)),
          pl.BlockSpec((BN, D), lambda i: (i, 0)),
          pl.BlockSpec((BN, 1), lambda i: (i, 0)),
          pl.BlockSpec((D, H), lambda i: (0, 0)),
          pl.BlockSpec((1, H), lambda i: (0, 0)),
          pl.BlockSpec((H, H), lambda i: (0, 0)),
          pl.BlockSpec((1, H), lambda i: (0, 0)),
          pl.BlockSpec((H, CPAD), lambda i: (0, 0)),
          pl.BlockSpec((1, CPAD), lambda i: (0, 0)),
      ],
      out_specs=pl.BlockSpec((BN, CPAD), lambda i: (i, 0)),
      out_shape=jax.ShapeDtypeStruct((NP, CPAD), jnp.float32),
  )(part, x_pad, deg2, W_conv, b_conv2, W_fc1, b_fc12, W_fc2p, b_fc2p2)


def kernel(x, edge_index, edge_attr, W_conv, b_conv, W_fc1, b_fc1, W_fc2,
           b_fc2):
  pad_spread = (jnp.arange(EP - E, dtype=jnp.int32) % (NP - N)) + N
  row = jnp.concatenate([edge_index[0], pad_spread]).reshape(ECH, CHUNK)
  col = jnp.concatenate([edge_index[1], pad_spread]).reshape(ECH, CHUNK)
  ew = jnp.pad(jnp.squeeze(edge_attr, -1), (0, EP - E)).reshape(ECH, CHUNK)
  x_pad = jnp.pad(x, ((0, NP - N), (0, 0)))

  deg, xs = _sc_degree(x_pad, col, ew)
  part = _sc_scatter(xs, row, col, ew)

  W_fc2p = jnp.pad(W_fc2, ((0, 0), (0, CPAD - C)))
  b_fc2p = jnp.pad(b_fc2, (0, CPAD - C))
  out = _tc_head(part, x_pad, deg.reshape(NP, 1),
                 W_conv.astype(jnp.bfloat16), b_conv.reshape(1, H),
                 W_fc1.astype(jnp.bfloat16), b_fc1.reshape(1, H),
                 W_fc2p.astype(jnp.bfloat16), b_fc2p.reshape(1, CPAD))
  return out[:N, :C]


# speedup vs baseline: 38.6620x; 1.0010x over previous
import jax
import jax.numpy as jnp
from jax import lax
from jax.experimental import pallas as pl
from jax.experimental.pallas import tpu as pltpu
from jax.experimental.pallas import tpu_sc as plsc

N = 10000
E = 320000
D = 128
H = 512
C = 40

NC = 2
NS = 16
NW = NC * NS
L = 16

NP = 10240
NODES_PER_TILE = NP // NS
NODES_PER_WORKER = NP // NW
NVEC = NODES_PER_TILE // L

CHUNK = 128
AGG_CHUNKS = 80
EDGES_PER_WORKER = AGG_CHUNKS * CHUNK
EP = EDGES_PER_WORKER * NW
ECH = EP // CHUNK
DEG_CHUNKS = ECH // NS

BLK = 16
NBLOCKS = AGG_CHUNKS // BLK
XROWS = 64
CPAD = 128


def _rsqrt16(d):
  i = plsc.bitcast(d, jnp.int32)
  y = plsc.bitcast(jnp.int32(0x5F3759DF) - (i >> 1), jnp.float32)
  for _ in range(3):
    y = y * (jnp.float32(1.5) - jnp.float32(0.5) * d * y * y)
  return y


def _deg_body(x_hbm, col_hbm, ew_hbm, deg_hbm, xs_hbm,
              degp_sh, deg_sh,
              dega_v, colD_v, ewD_v, tmp_v, degslice_v, diss_v, xbuf_v):
  cid = lax.axis_index("c")
  sid = lax.axis_index("s")
  wid = cid * NS + sid
  zero16 = jnp.zeros((L,), jnp.float32)

  @pl.loop(0, NP // L)
  def _(i):
    dega_v[pl.ds(i * L, L)] = zero16

  pltpu.sync_copy(col_hbm.at[pl.ds(sid * DEG_CHUNKS, DEG_CHUNKS)], colD_v)
  pltpu.sync_copy(ew_hbm.at[pl.ds(sid * DEG_CHUNKS, DEG_CHUNKS)], ewD_v)

  @pl.loop(0, DEG_CHUNKS)
  def _(c):
    for g in range(CHUNK // L):
      idx = colD_v[c, pl.ds(g * L, L)]
      w = ewD_v[c, pl.ds(g * L, L)]
      plsc.addupdate_scatter(dega_v, [idx], w)

  pltpu.sync_copy(dega_v, degp_sh.at[sid])
  plsc.subcore_barrier()

  nbase = sid * NODES_PER_TILE
  for j in range(NS):
    pltpu.sync_copy(degp_sh.at[j, pl.ds(nbase, NODES_PER_TILE)], tmp_v.at[j])

  @pl.loop(0, NVEC)
  def _(g):
    acc = jnp.full((L,), 1.0, jnp.float32)
    for j in range(NS):
      acc = acc + tmp_v[j, pl.ds(g * L, L)]
    degslice_v[pl.ds(g * L, L)] = acc

  pltpu.sync_copy(degslice_v, deg_sh.at[pl.ds(nbase, NODES_PER_TILE)])

  @pl.when(cid == 0)
  def _():
    pltpu.sync_copy(degslice_v, deg_hbm.at[pl.ds(nbase, NODES_PER_TILE)])

  plsc.subcore_barrier()

  rbase = wid * NODES_PER_WORKER
  pltpu.sync_copy(deg_sh.at[pl.ds(rbase, NODES_PER_WORKER)], diss_v)

  @pl.loop(0, NODES_PER_WORKER // L)
  def _(i):
    diss_v[pl.ds(i * L, L)] = _rsqrt16(diss_v[pl.ds(i * L, L)])

  @pl.loop(0, NODES_PER_WORKER // XROWS)
  def _(b):
    pltpu.sync_copy(x_hbm.at[pl.ds(rbase + b * XROWS, XROWS)], xbuf_v)

    @pl.loop(0, XROWS // L)
    def _(q):
      dv = diss_v[pl.ds(b * XROWS + q * L, L)]
      for lane in range(L):
        r = q * L + lane
        s = dv[lane]
        for g in range(D // L):
          xbuf_v[r, pl.ds(g * L, L)] = xbuf_v[r, pl.ds(g * L, L)] * s

    pltpu.sync_copy(xbuf_v, xs_hbm.at[pl.ds(rbase + b * XROWS, XROWS)])


@jax.jit
def _sc_degree(x_pad, col2d, ew2d):
  mesh = plsc.VectorSubcoreMesh(core_axis_name="c", subcore_axis_name="s")
  k = pl.kernel(
      _deg_body,
      out_type=(
          jax.ShapeDtypeStruct((NP,), jnp.float32),
          jax.ShapeDtypeStruct((NP, D), jnp.float32),
      ),
      mesh=mesh,
      compiler_params=pltpu.CompilerParams(needs_layout_passes=False),
      scratch_types=[
          pltpu.VMEM_SHARED((NS, NP), jnp.float32),
          pltpu.VMEM_SHARED((NP,), jnp.float32),
          pltpu.VMEM((NP,), jnp.float32),
          pltpu.VMEM((DEG_CHUNKS, CHUNK), jnp.int32),
          pltpu.VMEM((DEG_CHUNKS, CHUNK), jnp.float32),
          pltpu.VMEM((NS, NODES_PER_TILE), jnp.float32),
          pltpu.VMEM((NODES_PER_TILE,), jnp.float32),
          pltpu.VMEM((NODES_PER_WORKER,), jnp.float32),
          pltpu.VMEM((XROWS, D), jnp.float32),
      ],
  )
  return k(x_pad, col2d, ew2d)


def _agg_body(xs_hbm, row_hbm, col_hbm, ew_hbm, part_hbm,
              agg_sh, rowB_v, colB_v, ewB_v, rows2_v,
              gsem0, gsem1, ssem0, ssem1):
  cid = lax.axis_index("c")
  sid = lax.axis_index("s")
  wid = cid * NS + sid
  zero16 = jnp.zeros((L,), jnp.float32)

  @pl.loop(0, CHUNK)
  def _(r):
    for g in range(D // L):
      rows2_v[0, r, pl.ds(g * L, L)] = zero16

  for k in range(NODES_PER_TILE // CHUNK):
    pltpu.sync_copy(rows2_v.at[0],
                    agg_sh.at[pl.ds(sid * NODES_PER_TILE + k * CHUNK, CHUNK)])

  plsc.subcore_barrier()

  def scale(b, c):
    @pl.loop(0, CHUNK // L)
    def _(q):
      nv = ewB_v[c, pl.ds(q * L, L)]
      for lane in range(L):
        j = q * L + lane
        s = nv[lane]
        for g in range(D // L):
          rows2_v[b, j, pl.ds(g * L, L)] = rows2_v[b, j, pl.ds(g * L, L)] * s

  @pl.loop(0, NBLOCKS)
  def _(blk):
    cb = wid * AGG_CHUNKS + blk * BLK
    pltpu.sync_copy(row_hbm.at[pl.ds(cb, BLK)], rowB_v)
    pltpu.sync_copy(col_hbm.at[pl.ds(cb, BLK)], colB_v)
    pltpu.sync_copy(ew_hbm.at[pl.ds(cb, BLK)], ewB_v)

    pltpu.async_copy(xs_hbm.at[rowB_v.at[0]], rows2_v.at[0], gsem0)
    pltpu.async_copy(xs_hbm.at[rowB_v.at[1]], rows2_v.at[1], gsem1)

    @pl.loop(0, BLK // 2)
    def _(p):
      c0 = 2 * p
      c1 = 2 * p + 1
      pltpu.make_async_copy(xs_hbm.at[rowB_v.at[c0]], rows2_v.at[0],
                            gsem0).wait()
      scale(0, c0)
      s0 = pltpu.async_copy(rows2_v.at[0], agg_sh.at[colB_v.at[c0]], ssem0,
                            add=True)
      pltpu.make_async_copy(xs_hbm.at[rowB_v.at[c1]], rows2_v.at[1],
                            gsem1).wait()
      scale(1, c1)
      s1 = pltpu.async_copy(rows2_v.at[1], agg_sh.at[colB_v.at[c1]], ssem1,
                            add=True)
      c2 = jnp.minimum(c0 + 2, BLK - 1)
      c3 = jnp.minimum(c1 + 2, BLK - 1)
      s0.wait()

      @pl.when(p < BLK // 2 - 1)
      def _():
        pltpu.async_copy(xs_hbm.at[rowB_v.at[c2]], rows2_v.at[0], gsem0)

      s1.wait()

      @pl.when(p < BLK // 2 - 1)
      def _():
        pltpu.async_copy(xs_hbm.at[rowB_v.at[c3]], rows2_v.at[1], gsem1)

  plsc.subcore_barrier()

  nbase = sid * NODES_PER_TILE
  pltpu.sync_copy(agg_sh.at[pl.ds(nbase, NODES_PER_TILE)],
                  part_hbm.at[cid, pl.ds(nbase, NODES_PER_TILE)])


@jax.jit
def _sc_scatter(xs, row2d, col2d, ew2d):
  mesh = plsc.VectorSubcoreMesh(core_axis_name="c", subcore_axis_name="s")
  k = pl.kernel(
      _agg_body,
      out_type=jax.ShapeDtypeStruct((NC, NP, D), jnp.float32),
      mesh=mesh,
      compiler_params=pltpu.CompilerParams(needs_layout_passes=False),
      scratch_types=[
          pltpu.VMEM_SHARED((NP, D), jnp.float32),
          pltpu.VMEM((BLK, CHUNK), jnp.int32),
          pltpu.VMEM((BLK, CHUNK), jnp.int32),
          pltpu.VMEM((BLK, CHUNK), jnp.float32),
          pltpu.VMEM((2, CHUNK, D), jnp.float32),
          pltpu.SemaphoreType.DMA,
          pltpu.SemaphoreType.DMA,
          pltpu.SemaphoreType.DMA,
          pltpu.SemaphoreType.DMA,
      ],
  )
  return k(xs, row2d, col2d, ew2d)


def _tc_body(part_ref, x_ref, deg_ref, wc_ref, bc_ref, w1_ref, b1_ref,
             w2_ref, b2_ref, out_ref):
  deg = deg_ref[...]
  dis = lax.rsqrt(deg)
  agg = (part_ref[0] + part_ref[1]) * dis + x_ref[...] * (1.0 / deg)
  h = jnp.dot(agg.astype(jnp.bfloat16), wc_ref[...],
              preferred_element_type=jnp.float32)
  h = jnp.maximum(h + bc_ref[...], 0.0)
  h = jnp.dot(h.astype(jnp.bfloat16), w1_ref[...],
              preferred_element_type=jnp.float32)
  h = jnp.maximum(h + b1_ref[...], 0.0)
  logits = jnp.dot(h.astype(jnp.bfloat16), w2_ref[...],
                   preferred_element_type=jnp.float32)
  logits = logits + b2_ref[...]
  valid = lax.broadcasted_iota(jnp.int32, logits.shape, 1) < C
  masked = jnp.where(valid, logits, -jnp.inf)
  m = jnp.max(masked, axis=1, keepdims=True)
  s = jnp.sum(jnp.where(valid, jnp.exp(logits - m), 0.0), axis=1,
              keepdims=True)
  out_ref[...] = logits - m - jnp.log(s)


@jax.jit
def _tc_head(part, x_pad, deg2, W_conv, b_conv2, W_fc1, b_fc12, W_fc2p,
             b_fc2p2):
  BN = 256
  grid = (NP // BN,)
  return pl.pallas_call(
      _tc_body,
      grid=grid,
      in_specs=[
          pl.BlockSpec((NC, BN, D), lambda i: (0, i, 0)),
          pl.BlockSpec((BN, D), lambda i: (i, 0)),
          pl.BlockSpec((BN, 1), lambda i: (i, 0)),
          pl.BlockSpec((D, H), lambda i: (0, 0)),
          pl.BlockSpec((1, H), lambda i: (0, 0)),
          pl.BlockSpec((H, H), lambda i: (0, 0)),
          pl.BlockSpec((1, H), lambda i: (0, 0)),
          pl.BlockSpec((H, CPAD), lambda i: (0, 0)),
          pl.BlockSpec((1, CPAD), lambda i: (0, 0)),
      ],
      out_specs=pl.BlockSpec((BN, CPAD), lambda i: (i, 0)),
      out_shape=jax.ShapeDtypeStruct((NP, CPAD), jnp.float32),
  )(part, x_pad, deg2, W_conv, b_conv2, W_fc1, b_fc12, W_fc2p, b_fc2p2)


def kernel(x, edge_index, edge_attr, W_conv, b_conv, W_fc1, b_fc1, W_fc2,
           b_fc2):
  pad_spread = (jnp.arange(EP - E, dtype=jnp.int32) % (NP - N)) + N
  row = jnp.concatenate([edge_index[0], pad_spread]).reshape(ECH, CHUNK)
  col = jnp.concatenate([edge_index[1], pad_spread]).reshape(ECH, CHUNK)
  ew = jnp.pad(jnp.squeeze(edge_attr, -1), (0, EP - E)).reshape(ECH, CHUNK)
  x_pad = jnp.pad(x, ((0, NP - N), (0, 0)))

  deg, xs = _sc_degree(x_pad, col, ew)
  part = _sc_scatter(xs, row, col, ew)

  W_fc2p = jnp.pad(W_fc2, ((0, 0), (0, CPAD - C)))
  b_fc2p = jnp.pad(b_fc2, (0, CPAD - C))
  out = _tc_head(part, x_pad, deg.reshape(NP, 1),
                 W_conv.astype(jnp.bfloat16), b_conv.reshape(1, H),
                 W_fc1.astype(jnp.bfloat16), b_fc1.reshape(1, H),
                 W_fc2p.astype(jnp.bfloat16), b_fc2p.reshape(1, CPAD))
  return out[:N, :C]
